# initial kernel scaffold (unmeasured)
import jax
import jax.numpy as jnp
from jax import lax
from jax.experimental import pallas as pl
from jax.experimental.pallas import tpu as pltpu

N_DEV = 16
B = 2
SQ = 128
SKV = 2048
HQ = 64
DH = 64
H_LOC = HQ // N_DEV
SKV_LOC = SKV // N_DEV
DM = 512
BLK = 64

BF16 = jnp.bfloat16
F32 = jnp.float32


def _mask_add():
    qb = lax.broadcasted_iota(jnp.int32, (SQ, SKV), 0) // BLK
    kb = lax.broadcasted_iota(jnp.int32, (SQ, SKV), 1) // BLK
    keep = (qb == kb) | (kb == 0) | (((qb + kb) % 3) == 0)
    return jnp.where(keep, jnp.float32(0.0), jnp.float32(-1e9))


def kernel(x, Wq, K_ext, V_ext, Wo):
    def body(x_ref, wq_ref, k_ref, v_ref, wo_ref, out_ref,
             kvs_ref, kvbuf_ref, pstage_ref, pbuf_ref,
             send1, recv1, send2, recv2, loc_sem):
        my = lax.axis_index("i")

        kbf = k_ref[...].astype(BF16)
        vbf = v_ref[...].astype(BF16)
        for g in range(N_DEV):
            kvs_ref[g, 0] = kbf[:, :, g * H_LOC:(g + 1) * H_LOC, :]
            kvs_ref[g, 1] = vbf[:, :, g * H_LOC:(g + 1) * H_LOC, :]

        loc1 = pltpu.make_async_copy(kvs_ref.at[my], kvbuf_ref.at[my],
                                     loc_sem.at[0])
        loc1.start()

        for d in range(1, N_DEV):
            t = (my + d) % N_DEV
            rdma = pltpu.make_async_remote_copy(
                src_ref=kvs_ref.at[t],
                dst_ref=kvbuf_ref.at[my],
                send_sem=send1.at[d],
                recv_sem=recv1.at[my],
                device_id=(t,),
                device_id_type=pl.DeviceIdType.MESH,
            )
            rdma.start()

        wqv = wq_ref[...].astype(BF16)
        q_b = []
        for b in range(B):
            xb = x_ref[b].astype(BF16)
            q = lax.dot_general(xb, wqv, (((1,), (0,)), ((), ())),
                                preferred_element_type=F32)
            q_b.append(q)
        madd = _mask_add()

        loc1.wait()
        for d in range(1, N_DEV):
            s = (my - d + N_DEV) % N_DEV
            pltpu.make_async_remote_copy(
                src_ref=kvs_ref.at[s],
                dst_ref=kvbuf_ref.at[s],
                send_sem=send1.at[d],
                recv_sem=recv1.at[s],
                device_id=(s,),
                device_id_type=pl.DeviceIdType.MESH,
            ).wait_recv()

        wov = wo_ref[...].astype(BF16)
        for b in range(B):
            ctx_h = []
            for h in range(H_LOC):
                qbh = q_b[b][:, h * DH:(h + 1) * DH].astype(BF16)
                kbh = kvbuf_ref[:, 0, b, :, h, :].reshape(SKV, DH)
                s = lax.dot_general(qbh, kbh, (((1,), (1,)), ((), ())),
                                    preferred_element_type=F32)
                s = s * 0.125 + madd
                m = jnp.max(s, axis=1, keepdims=True)
                w = jnp.exp(s - m)
                w = w / jnp.sum(w, axis=1, keepdims=True)
                vbh = kvbuf_ref[:, 1, b, :, h, :].reshape(SKV, DH)
                ctx = lax.dot_general(w.astype(BF16), vbh,
                                      (((1,), (0,)), ((), ())),
                                      preferred_element_type=F32)
                ctx_h.append(ctx)
            ctx_b = jnp.concatenate(ctx_h, axis=1).astype(BF16)
            pstage_ref[b] = lax.dot_general(
                ctx_b, wov, (((1,), (0,)), ((), ())),
                preferred_element_type=F32).astype(BF16)

        loc2 = pltpu.make_async_copy(pstage_ref, pbuf_ref.at[my],
                                     loc_sem.at[1])
        loc2.start()

        for d in range(1, N_DEV):
            t = (my + d) % N_DEV
            rdma = pltpu.make_async_remote_copy(
                src_ref=pstage_ref,
                dst_ref=pbuf_ref.at[my],
                send_sem=send2.at[d],
                recv_sem=recv2.at[my],
                device_id=(t,),
                device_id_type=pl.DeviceIdType.MESH,
            )
            rdma.start()

        for d in range(1, N_DEV):
            t = (my + d) % N_DEV
            pltpu.make_async_remote_copy(
                src_ref=kvs_ref.at[t],
                dst_ref=kvbuf_ref.at[my],
                send_sem=send1.at[d],
                recv_sem=recv1.at[my],
                device_id=(t,),
                device_id_type=pl.DeviceIdType.MESH,
            ).wait_send()

        loc2.wait()
        for d in range(1, N_DEV):
            s = (my - d + N_DEV) % N_DEV
            pltpu.make_async_remote_copy(
                src_ref=pstage_ref,
                dst_ref=pbuf_ref.at[s],
                send_sem=send2.at[d],
                recv_sem=recv2.at[s],
                device_id=(s,),
                device_id_type=pl.DeviceIdType.MESH,
            ).wait_recv()

        acc = pbuf_ref[...].astype(F32)
        out_ref[...] = jnp.sum(acc, axis=0)

        for d in range(1, N_DEV):
            t = (my + d) % N_DEV
            pltpu.make_async_remote_copy(
                src_ref=pstage_ref,
                dst_ref=pbuf_ref.at[my],
                send_sem=send2.at[d],
                recv_sem=recv2.at[my],
                device_id=(t,),
                device_id_type=pl.DeviceIdType.MESH,
            ).wait_send()

    return pl.pallas_call(
        body,
        out_shape=jax.ShapeDtypeStruct((B, SQ, DM), F32),
        in_specs=[pl.BlockSpec(memory_space=pltpu.VMEM)] * 5,
        out_specs=pl.BlockSpec(memory_space=pltpu.VMEM),
        scratch_shapes=[
            pltpu.VMEM((N_DEV, 2, B, SKV_LOC, H_LOC, DH), BF16),
            pltpu.VMEM((N_DEV, 2, B, SKV_LOC, H_LOC, DH), BF16),
            pltpu.VMEM((B, SQ, DM), BF16),
            pltpu.VMEM((N_DEV, B, SQ, DM), BF16),
            pltpu.SemaphoreType.DMA((N_DEV,)),
            pltpu.SemaphoreType.DMA((N_DEV,)),
            pltpu.SemaphoreType.DMA((N_DEV,)),
            pltpu.SemaphoreType.DMA((N_DEV,)),
            pltpu.SemaphoreType.DMA((2,)),
        ],
        compiler_params=pltpu.CompilerParams(collective_id=0),
    )(x, Wq, K_ext, V_ext, Wo)


# baseline (device time: 202254 ns/iter reference)
import jax
import jax.numpy as jnp
from jax import lax
from jax.experimental import pallas as pl
from jax.experimental.pallas import tpu as pltpu

N_DEV = 16
B = 2
SQ = 128
SKV = 2048
HQ = 64
DH = 64
H_LOC = HQ // N_DEV
SKV_LOC = SKV // N_DEV
DM = 512
BLK = 64

BF16 = jnp.bfloat16
F32 = jnp.float32


def _mask_add():
    qb = lax.broadcasted_iota(jnp.int32, (SQ, SKV), 0) // BLK
    kb = lax.broadcasted_iota(jnp.int32, (SQ, SKV), 1) // BLK
    keep = (qb == kb) | (kb == 0) | (((qb + kb) % 3) == 0)
    return jnp.where(keep, jnp.float32(0.0), jnp.float32(-1e9))


def kernel(x, Wq, K_ext, V_ext, Wo):
    def body(x_ref, wq_ref, k_ref, v_ref, wo_ref, out_ref,
             kvs_ref, kvbuf_ref, pstage_ref, pbuf_ref,
             send1, recv1, send2, recv2, loc_sem):
        my = lax.axis_index("i")

        barrier = pltpu.get_barrier_semaphore()
        for d in range(1, N_DEV):
            t = (my + d) % N_DEV
            pltpu.semaphore_signal(
                barrier, inc=1, device_id=(t,),
                device_id_type=pl.DeviceIdType.MESH)
        pltpu.semaphore_wait(barrier, N_DEV - 1)

        kbf = k_ref[...].astype(BF16)
        vbf = v_ref[...].astype(BF16)
        for g in range(N_DEV):
            kvs_ref[g, 0] = kbf[:, :, g * H_LOC:(g + 1) * H_LOC, :]
            kvs_ref[g, 1] = vbf[:, :, g * H_LOC:(g + 1) * H_LOC, :]

        loc1 = pltpu.make_async_copy(kvs_ref.at[my], kvbuf_ref.at[my],
                                     loc_sem.at[0])
        loc1.start()

        for d in range(1, N_DEV):
            t = (my + d) % N_DEV
            rdma = pltpu.make_async_remote_copy(
                src_ref=kvs_ref.at[t],
                dst_ref=kvbuf_ref.at[my],
                send_sem=send1.at[d],
                recv_sem=recv1.at[my],
                device_id=(t,),
                device_id_type=pl.DeviceIdType.MESH,
            )
            rdma.start()

        wqv = wq_ref[...].astype(BF16)
        q_b = []
        for b in range(B):
            xb = x_ref[b].astype(BF16)
            q = lax.dot_general(xb, wqv, (((1,), (0,)), ((), ())),
                                preferred_element_type=F32)
            q_b.append(q)
        madd = _mask_add()

        loc1.wait()
        for d in range(1, N_DEV):
            s = (my - d + N_DEV) % N_DEV
            pltpu.make_async_remote_copy(
                src_ref=kvs_ref.at[s],
                dst_ref=kvbuf_ref.at[s],
                send_sem=send1.at[d],
                recv_sem=recv1.at[s],
                device_id=(s,),
                device_id_type=pl.DeviceIdType.MESH,
            ).wait_recv()

        wov = wo_ref[...].astype(BF16)
        for b in range(B):
            ctx_h = []
            for h in range(H_LOC):
                qbh = q_b[b][:, h * DH:(h + 1) * DH].astype(BF16)
                kbh = kvbuf_ref[:, 0, b, :, h, :].reshape(SKV, DH)
                s = lax.dot_general(qbh, kbh, (((1,), (1,)), ((), ())),
                                    preferred_element_type=F32)
                s = s * 0.125 + madd
                m = jnp.max(s, axis=1, keepdims=True)
                w = jnp.exp(s - m)
                w = w / jnp.sum(w, axis=1, keepdims=True)
                vbh = kvbuf_ref[:, 1, b, :, h, :].reshape(SKV, DH)
                ctx = lax.dot_general(w.astype(BF16), vbh,
                                      (((1,), (0,)), ((), ())),
                                      preferred_element_type=F32)
                ctx_h.append(ctx)
            ctx_b = jnp.concatenate(ctx_h, axis=1).astype(BF16)
            pstage_ref[b] = lax.dot_general(
                ctx_b, wov, (((1,), (0,)), ((), ())),
                preferred_element_type=F32).astype(BF16)

        loc2 = pltpu.make_async_copy(pstage_ref, pbuf_ref.at[my],
                                     loc_sem.at[1])
        loc2.start()

        for d in range(1, N_DEV):
            t = (my + d) % N_DEV
            rdma = pltpu.make_async_remote_copy(
                src_ref=pstage_ref,
                dst_ref=pbuf_ref.at[my],
                send_sem=send2.at[d],
                recv_sem=recv2.at[my],
                device_id=(t,),
                device_id_type=pl.DeviceIdType.MESH,
            )
            rdma.start()

        for d in range(1, N_DEV):
            t = (my + d) % N_DEV
            pltpu.make_async_remote_copy(
                src_ref=kvs_ref.at[t],
                dst_ref=kvbuf_ref.at[my],
                send_sem=send1.at[d],
                recv_sem=recv1.at[my],
                device_id=(t,),
                device_id_type=pl.DeviceIdType.MESH,
            ).wait_send()

        loc2.wait()
        for d in range(1, N_DEV):
            s = (my - d + N_DEV) % N_DEV
            pltpu.make_async_remote_copy(
                src_ref=pstage_ref,
                dst_ref=pbuf_ref.at[s],
                send_sem=send2.at[d],
                recv_sem=recv2.at[s],
                device_id=(s,),
                device_id_type=pl.DeviceIdType.MESH,
            ).wait_recv()

        acc = pbuf_ref[...].astype(F32)
        out_ref[...] = jnp.sum(acc, axis=0)

        for d in range(1, N_DEV):
            t = (my + d) % N_DEV
            pltpu.make_async_remote_copy(
                src_ref=pstage_ref,
                dst_ref=pbuf_ref.at[my],
                send_sem=send2.at[d],
                recv_sem=recv2.at[my],
                device_id=(t,),
                device_id_type=pl.DeviceIdType.MESH,
            ).wait_send()

    return pl.pallas_call(
        body,
        out_shape=jax.ShapeDtypeStruct((B, SQ, DM), F32),
        in_specs=[pl.BlockSpec(memory_space=pltpu.VMEM)] * 5,
        out_specs=pl.BlockSpec(memory_space=pltpu.VMEM),
        scratch_shapes=[
            pltpu.VMEM((N_DEV, 2, B, SKV_LOC, H_LOC, DH), BF16),
            pltpu.VMEM((N_DEV, 2, B, SKV_LOC, H_LOC, DH), BF16),
            pltpu.VMEM((B, SQ, DM), BF16),
            pltpu.VMEM((N_DEV, B, SQ, DM), BF16),
            pltpu.SemaphoreType.DMA((N_DEV,)),
            pltpu.SemaphoreType.DMA((N_DEV,)),
            pltpu.SemaphoreType.DMA((N_DEV,)),
            pltpu.SemaphoreType.DMA((N_DEV,)),
            pltpu.SemaphoreType.DMA((2,)),
        ],
        compiler_params=pltpu.CompilerParams(collective_id=0),
    )(x, Wq, K_ext, V_ext, Wo)


# device time: 105834 ns/iter; 1.9110x vs baseline; 1.9110x over previous
import jax
import jax.numpy as jnp
from jax import lax
from jax.experimental import pallas as pl
from jax.experimental.pallas import tpu as pltpu

N_DEV = 16
LOG2_DEV = 4
B = 2
SQ = 128
SKV = 2048
HQ = 64
DH = 64
H_LOC = HQ // N_DEV
SKV_LOC = SKV // N_DEV
GRP = H_LOC * DH
DM = 512
BLK = 64

BF16 = jnp.bfloat16
F32 = jnp.float32


def _mask_add():
    qb = lax.broadcasted_iota(jnp.int32, (SQ, SKV), 0) // BLK
    kb = lax.broadcasted_iota(jnp.int32, (SQ, SKV), 1) // BLK
    keep = (qb == kb) | (kb == 0) | (((qb + kb) % 3) == 0)
    return jnp.where(keep, jnp.float32(0.0), jnp.float32(-1e9))


def kernel(x, Wq, K_ext, V_ext, Wo):
    def body(x_ref, wq_ref, k_ref, v_ref, wo_ref, out_ref,
             kvbf_ref, kvbuf_ref, pstage_ref, pbuf2_ref,
             send1, recv1, send2, recv2, loc_sem):
        my = lax.axis_index("i")

        barrier = pltpu.get_barrier_semaphore()
        for d in range(1, N_DEV):
            t = (my + d) % N_DEV
            pltpu.semaphore_signal(
                barrier, inc=1, device_id=(t,),
                device_id_type=pl.DeviceIdType.MESH)

        kvbf_ref[0] = k_ref[...].reshape(B, SKV_LOC, HQ * DH).astype(BF16)
        kvbf_ref[1] = v_ref[...].reshape(B, SKV_LOC, HQ * DH).astype(BF16)

        loc1 = pltpu.make_async_copy(
            kvbf_ref.at[:, :, :, pl.ds(my * GRP, GRP)],
            kvbuf_ref.at[my], loc_sem.at[0])
        loc1.start()

        pltpu.semaphore_wait(barrier, N_DEV - 1)
        for d in range(1, N_DEV):
            t = (my + d) % N_DEV
            pltpu.make_async_remote_copy(
                src_ref=kvbf_ref.at[:, :, :, pl.ds(t * GRP, GRP)],
                dst_ref=kvbuf_ref.at[my],
                send_sem=send1.at[d],
                recv_sem=recv1.at[my],
                device_id=(t,),
                device_id_type=pl.DeviceIdType.MESH,
            ).start()

        wqv = wq_ref[...].astype(BF16)
        q_b = []
        for b in range(B):
            xb = x_ref[b].astype(BF16)
            q = lax.dot_general(xb, wqv, (((1,), (0,)), ((), ())),
                                preferred_element_type=F32)
            q_b.append(q * 0.125)
        madd = _mask_add()

        loc1.wait()
        for d in range(1, N_DEV):
            s = (my - d + N_DEV) % N_DEV
            pltpu.make_async_remote_copy(
                src_ref=kvbuf_ref.at[s],
                dst_ref=kvbuf_ref.at[s],
                send_sem=send1.at[d],
                recv_sem=recv1.at[s],
                device_id=(s,),
                device_id_type=pl.DeviceIdType.MESH,
            ).wait_recv()

        wov = wo_ref[...].astype(BF16)
        acc = []
        for b in range(B):
            ctx_h = []
            for h in range(H_LOC):
                qbh = q_b[b][:, h * DH:(h + 1) * DH].astype(BF16)
                kbh = kvbuf_ref[:, 0, b, :, h * DH:(h + 1) * DH].reshape(SKV, DH)
                s = lax.dot_general(qbh, kbh, (((1,), (1,)), ((), ())),
                                    preferred_element_type=F32)
                s = s + madd
                w = jnp.exp(s)
                w = w / jnp.sum(w, axis=1, keepdims=True)
                vbh = kvbuf_ref[:, 1, b, :, h * DH:(h + 1) * DH].reshape(SKV, DH)
                ctx_h.append(lax.dot_general(w.astype(BF16), vbh,
                                             (((1,), (0,)), ((), ())),
                                             preferred_element_type=F32))
            ctx_b = jnp.concatenate(ctx_h, axis=1).astype(BF16)
            acc.append(lax.dot_general(ctx_b, wov, (((1,), (0,)), ((), ())),
                                       preferred_element_type=F32))

        for r in range(LOG2_DEV):
            partner = jnp.bitwise_xor(my, 1 << r)
            for b in range(B):
                pstage_ref[b] = acc[b].astype(BF16)
            rdma = pltpu.make_async_remote_copy(
                src_ref=pstage_ref,
                dst_ref=pbuf2_ref.at[r],
                send_sem=send2.at[r],
                recv_sem=recv2.at[r],
                device_id=(partner,),
                device_id_type=pl.DeviceIdType.MESH,
            )
            rdma.start()
            rdma.wait_recv()
            got = pbuf2_ref[r].astype(F32)
            acc = [acc[b] + got[b] for b in range(B)]
            rdma.wait_send()

        for b in range(B):
            out_ref[b] = acc[b]

        for d in range(1, N_DEV):
            t = (my + d) % N_DEV
            pltpu.make_async_remote_copy(
                src_ref=kvbf_ref.at[:, :, :, pl.ds(t * GRP, GRP)],
                dst_ref=kvbuf_ref.at[my],
                send_sem=send1.at[d],
                recv_sem=recv1.at[my],
                device_id=(t,),
                device_id_type=pl.DeviceIdType.MESH,
            ).wait_send()

    return pl.pallas_call(
        body,
        out_shape=jax.ShapeDtypeStruct((B, SQ, DM), F32),
        in_specs=[pl.BlockSpec(memory_space=pltpu.VMEM)] * 5,
        out_specs=pl.BlockSpec(memory_space=pltpu.VMEM),
        scratch_shapes=[
            pltpu.VMEM((2, B, SKV_LOC, HQ * DH), BF16),
            pltpu.VMEM((N_DEV, 2, B, SKV_LOC, GRP), BF16),
            pltpu.VMEM((B, SQ, DM), BF16),
            pltpu.VMEM((LOG2_DEV, B, SQ, DM), BF16),
            pltpu.SemaphoreType.DMA((N_DEV,)),
            pltpu.SemaphoreType.DMA((N_DEV,)),
            pltpu.SemaphoreType.DMA((LOG2_DEV,)),
            pltpu.SemaphoreType.DMA((LOG2_DEV,)),
            pltpu.SemaphoreType.DMA((1,)),
        ],
        compiler_params=pltpu.CompilerParams(collective_id=0),
    )(x, Wq, K_ext, V_ext, Wo)


# device time: 98219 ns/iter; 2.0592x vs baseline; 1.0775x over previous
import numpy as np

import jax
import jax.numpy as jnp
from jax import lax
from jax.experimental import pallas as pl
from jax.experimental.pallas import tpu as pltpu

N_DEV = 16
LOG2_DEV = 4
B = 2
SQ = 128
SKV = 2048
HQ = 64
DH = 64
H_LOC = HQ // N_DEV
SKV_LOC = SKV // N_DEV
GRP = H_LOC * DH
DM = 512
BLK = 64
N_BLK = SKV // BLK

_NEEDED = [g for g in range(N_BLK) if g == 1 or g % 3 != 1]
SKVC = len(_NEEDED) * BLK
_KB_OF_ROW = np.repeat(np.array(_NEEDED, np.int32), BLK)

BF16 = jnp.bfloat16
F32 = jnp.float32


def _mask_add():
    qb = lax.broadcasted_iota(jnp.int32, (SQ, SKVC), 0) // BLK
    c = lax.broadcasted_iota(jnp.int32, (SQ, SKVC), 1) // BLK
    kb = jnp.where(c < 2, c, 2 + 3 * ((c - 2) // 2) + (c - 2) % 2)
    keep = (qb == kb) | (kb == 0) | (((qb + kb) % 3) == 0)
    return jnp.where(keep, jnp.float32(0.0), jnp.float32(-1e9))


def kernel(x, Wq, K_ext, V_ext, Wo):
    def body(x_ref, wq_ref, k_ref, v_ref, wo_ref, out_ref,
             kvbf_ref, kvbuf_ref, pstage_ref, pbuf2_ref,
             send1, recv1, send2, recv2, loc_sem):
        my = lax.axis_index("i")

        def blk_needed(blk):
            return (blk == 1) | (blk % 3 != 1)

        def blk_dstoff(blk):
            nex = jnp.clip((blk - 2) // 3, 0, N_BLK - len(_NEEDED))
            return (blk - nex) * BLK

        barrier = pltpu.get_barrier_semaphore()
        for d in range(1, N_DEV):
            t = (my + d) % N_DEV
            pltpu.semaphore_signal(
                barrier, inc=1, device_id=(t,),
                device_id_type=pl.DeviceIdType.MESH)

        kvbf_ref[0] = k_ref[...].reshape(B, SKV_LOC, HQ * DH).astype(BF16)
        kvbf_ref[1] = v_ref[...].reshape(B, SKV_LOC, HQ * DH).astype(BF16)

        for j in range(2):
            blk = 2 * my + j

            @pl.when(blk_needed(blk))
            def _(j=j, blk=blk):
                pltpu.make_async_copy(
                    kvbf_ref.at[:, :, pl.ds(j * BLK, BLK),
                                pl.ds(my * GRP, GRP)],
                    kvbuf_ref.at[:, :, pl.ds(blk_dstoff(blk), BLK), :],
                    loc_sem.at[j]).start()

        pltpu.semaphore_wait(barrier, N_DEV - 1)
        for j in range(2):
            blk = 2 * my + j

            @pl.when(blk_needed(blk))
            def _(j=j, blk=blk):
                for d in range(1, N_DEV):
                    t = (my + d) % N_DEV
                    pltpu.make_async_remote_copy(
                        src_ref=kvbf_ref.at[:, :, pl.ds(j * BLK, BLK),
                                            pl.ds(t * GRP, GRP)],
                        dst_ref=kvbuf_ref.at[:, :, pl.ds(blk_dstoff(blk),
                                                         BLK), :],
                        send_sem=send1.at[2 * d + j],
                        recv_sem=recv1.at[2 * my + j],
                        device_id=(t,),
                        device_id_type=pl.DeviceIdType.MESH,
                    ).start()

        wqv = wq_ref[...].astype(BF16)
        q_b = []
        for b in range(B):
            xb = x_ref[b].astype(BF16)
            q = lax.dot_general(xb, wqv, (((1,), (0,)), ((), ())),
                                preferred_element_type=F32)
            q_b.append(q * 0.125)
        madd = _mask_add()

        for j in range(2):
            blk = 2 * my + j

            @pl.when(blk_needed(blk))
            def _(j=j, blk=blk):
                pltpu.make_async_copy(
                    kvbf_ref.at[:, :, pl.ds(j * BLK, BLK),
                                pl.ds(my * GRP, GRP)],
                    kvbuf_ref.at[:, :, pl.ds(blk_dstoff(blk), BLK), :],
                    loc_sem.at[j]).wait()

        for d in range(1, N_DEV):
            s = (my - d + N_DEV) % N_DEV
            for j in range(2):
                blk = 2 * s + j

                @pl.when(blk_needed(blk))
                def _(s=s, j=j, blk=blk, d=d):
                    pltpu.make_async_remote_copy(
                        src_ref=kvbf_ref.at[:, :, pl.ds(j * BLK, BLK),
                                            pl.ds(s * GRP, GRP)],
                        dst_ref=kvbuf_ref.at[:, :, pl.ds(blk_dstoff(blk),
                                                         BLK), :],
                        send_sem=send1.at[2 * d + j],
                        recv_sem=recv1.at[2 * s + j],
                        device_id=(s,),
                        device_id_type=pl.DeviceIdType.MESH,
                    ).wait_recv()

        wov = wo_ref[...].astype(BF16)
        acc = []
        for b in range(B):
            ctx_h = []
            for h in range(H_LOC):
                qbh = q_b[b][:, h * DH:(h + 1) * DH].astype(BF16)
                kbh = kvbuf_ref[0, b, :, h * DH:(h + 1) * DH]
                s = lax.dot_general(qbh, kbh, (((1,), (1,)), ((), ())),
                                    preferred_element_type=F32)
                w = jnp.exp(s + madd)
                l = jnp.sum(w, axis=1, keepdims=True)
                vbh = kvbuf_ref[1, b, :, h * DH:(h + 1) * DH]
                ctx = lax.dot_general(w.astype(BF16), vbh,
                                      (((1,), (0,)), ((), ())),
                                      preferred_element_type=F32)
                ctx_h.append(ctx / l)
            ctx_b = jnp.concatenate(ctx_h, axis=1).astype(BF16)
            acc.append(lax.dot_general(ctx_b, wov, (((1,), (0,)), ((), ())),
                                       preferred_element_type=F32))

        for r in range(LOG2_DEV):
            partner = jnp.bitwise_xor(my, 1 << r)
            for b in range(B):
                pstage_ref[b] = acc[b].astype(BF16)
            rdma = pltpu.make_async_remote_copy(
                src_ref=pstage_ref,
                dst_ref=pbuf2_ref.at[r],
                send_sem=send2.at[r],
                recv_sem=recv2.at[r],
                device_id=(partner,),
                device_id_type=pl.DeviceIdType.MESH,
            )
            rdma.start()
            rdma.wait_recv()
            got = pbuf2_ref[r].astype(F32)
            acc = [acc[b] + got[b] for b in range(B)]
            rdma.wait_send()

        for b in range(B):
            out_ref[b] = acc[b]

        for j in range(2):
            blk = 2 * my + j

            @pl.when(blk_needed(blk))
            def _(j=j, blk=blk):
                for d in range(1, N_DEV):
                    t = (my + d) % N_DEV
                    pltpu.make_async_remote_copy(
                        src_ref=kvbf_ref.at[:, :, pl.ds(j * BLK, BLK),
                                            pl.ds(t * GRP, GRP)],
                        dst_ref=kvbuf_ref.at[:, :, pl.ds(blk_dstoff(blk),
                                                         BLK), :],
                        send_sem=send1.at[2 * d + j],
                        recv_sem=recv1.at[2 * my + j],
                        device_id=(t,),
                        device_id_type=pl.DeviceIdType.MESH,
                    ).wait_send()

    return pl.pallas_call(
        body,
        out_shape=jax.ShapeDtypeStruct((B, SQ, DM), F32),
        in_specs=[pl.BlockSpec(memory_space=pltpu.VMEM)] * 5,
        out_specs=pl.BlockSpec(memory_space=pltpu.VMEM),
        scratch_shapes=[
            pltpu.VMEM((2, B, SKV_LOC, HQ * DH), BF16),
            pltpu.VMEM((2, B, SKVC, GRP), BF16),
            pltpu.VMEM((B, SQ, DM), BF16),
            pltpu.VMEM((LOG2_DEV, B, SQ, DM), BF16),
            pltpu.SemaphoreType.DMA((2 * N_DEV,)),
            pltpu.SemaphoreType.DMA((2 * N_DEV,)),
            pltpu.SemaphoreType.DMA((LOG2_DEV,)),
            pltpu.SemaphoreType.DMA((LOG2_DEV,)),
            pltpu.SemaphoreType.DMA((2,)),
        ],
        compiler_params=pltpu.CompilerParams(collective_id=0),
    )(x, Wq, K_ext, V_ext, Wo)


# device time: 86585 ns/iter; 2.3359x vs baseline; 1.1344x over previous
import numpy as np

import jax
import jax.numpy as jnp
from jax import lax
from jax.experimental import pallas as pl
from jax.experimental.pallas import tpu as pltpu

N_DEV = 16
LOG2_DEV = 4
B = 2
SQ = 128
SKV = 2048
HQ = 64
DH = 64
H_LOC = HQ // N_DEV
SKV_LOC = SKV // N_DEV
GRP = H_LOC * DH
DM = 512
BLK = 64
N_BLK = SKV // BLK

_NEEDED = [g for g in range(N_BLK) if g == 1 or g % 3 != 1]
SKVC = len(_NEEDED) * BLK
_KB_OF_ROW = np.repeat(np.array(_NEEDED, np.int32), BLK)

BF16 = jnp.bfloat16
F32 = jnp.float32


def _mask_add():
    qb = lax.broadcasted_iota(jnp.int32, (SQ, SKVC), 0) // BLK
    c = lax.broadcasted_iota(jnp.int32, (SQ, SKVC), 1) // BLK
    kb = jnp.where(c < 2, c, 2 + 3 * ((c - 2) // 2) + (c - 2) % 2)
    keep = (qb == kb) | (kb == 0) | (((qb + kb) % 3) == 0)
    return jnp.where(keep, jnp.float32(0.0), jnp.float32(-1e9))


def kernel(x, Wq, K_ext, V_ext, Wo):
    def body(x_ref, wq_ref, k_ref, v_ref, wo_ref, out_ref,
             kvbf_ref, kvbuf_ref, pstage_ref, rsbuf_ref, agstage_ref,
             agbuf_ref, send1, recv1, send_rs, recv_rs, send_ag, recv_ag,
             loc_sem):
        my = lax.axis_index("i")

        def blk_needed(blk):
            return (blk == 1) | (blk % 3 != 1)

        def blk_dstoff(blk):
            nex = jnp.clip((blk - 2) // 3, 0, N_BLK - len(_NEEDED))
            return (blk - nex) * BLK

        barrier = pltpu.get_barrier_semaphore()
        for d in range(1, N_DEV):
            t = (my + d) % N_DEV
            pltpu.semaphore_signal(
                barrier, inc=1, device_id=(t,),
                device_id_type=pl.DeviceIdType.MESH)

        kvbf_ref[0] = k_ref[...].reshape(B, SKV_LOC, HQ * DH).astype(BF16)
        kvbf_ref[1] = v_ref[...].reshape(B, SKV_LOC, HQ * DH).astype(BF16)

        for j in range(2):
            blk = 2 * my + j

            @pl.when(blk_needed(blk))
            def _(j=j, blk=blk):
                pltpu.make_async_copy(
                    kvbf_ref.at[:, :, pl.ds(j * BLK, BLK),
                                pl.ds(my * GRP, GRP)],
                    kvbuf_ref.at[:, :, pl.ds(blk_dstoff(blk), BLK), :],
                    loc_sem.at[j]).start()

        pltpu.semaphore_wait(barrier, N_DEV - 1)
        for j in range(2):
            blk = 2 * my + j

            @pl.when(blk_needed(blk))
            def _(j=j, blk=blk):
                for d in range(1, N_DEV):
                    t = (my + d) % N_DEV
                    pltpu.make_async_remote_copy(
                        src_ref=kvbf_ref.at[:, :, pl.ds(j * BLK, BLK),
                                            pl.ds(t * GRP, GRP)],
                        dst_ref=kvbuf_ref.at[:, :, pl.ds(blk_dstoff(blk),
                                                         BLK), :],
                        send_sem=send1.at[2 * d + j],
                        recv_sem=recv1.at[2 * my + j],
                        device_id=(t,),
                        device_id_type=pl.DeviceIdType.MESH,
                    ).start()

        wqv = wq_ref[...].astype(BF16)
        q_b = []
        for b in range(B):
            xb = x_ref[b].astype(BF16)
            q = lax.dot_general(xb, wqv, (((1,), (0,)), ((), ())),
                                preferred_element_type=F32)
            q_b.append(q * 0.125)
        madd = _mask_add()

        for j in range(2):
            blk = 2 * my + j

            @pl.when(blk_needed(blk))
            def _(j=j, blk=blk):
                pltpu.make_async_copy(
                    kvbf_ref.at[:, :, pl.ds(j * BLK, BLK),
                                pl.ds(my * GRP, GRP)],
                    kvbuf_ref.at[:, :, pl.ds(blk_dstoff(blk), BLK), :],
                    loc_sem.at[j]).wait()

        for d in range(1, N_DEV):
            s = (my - d + N_DEV) % N_DEV
            for j in range(2):
                blk = 2 * s + j

                @pl.when(blk_needed(blk))
                def _(s=s, j=j, blk=blk, d=d):
                    pltpu.make_async_remote_copy(
                        src_ref=kvbf_ref.at[:, :, pl.ds(j * BLK, BLK),
                                            pl.ds(s * GRP, GRP)],
                        dst_ref=kvbuf_ref.at[:, :, pl.ds(blk_dstoff(blk),
                                                         BLK), :],
                        send_sem=send1.at[2 * d + j],
                        recv_sem=recv1.at[2 * s + j],
                        device_id=(s,),
                        device_id_type=pl.DeviceIdType.MESH,
                    ).wait_recv()

        wov = wo_ref[...].astype(BF16)
        acc = []
        for b in range(B):
            ctx_h = []
            for h in range(H_LOC):
                qbh = q_b[b][:, h * DH:(h + 1) * DH].astype(BF16)
                kbh = kvbuf_ref[0, b, :, h * DH:(h + 1) * DH]
                s = lax.dot_general(qbh, kbh, (((1,), (1,)), ((), ())),
                                    preferred_element_type=F32)
                w = jnp.exp(s + madd)
                l = jnp.sum(w, axis=1, keepdims=True)
                vbh = kvbuf_ref[1, b, :, h * DH:(h + 1) * DH]
                ctx = lax.dot_general(w.astype(BF16), vbh,
                                      (((1,), (0,)), ((), ())),
                                      preferred_element_type=F32)
                ctx_h.append(ctx / l)
            ctx_b = jnp.concatenate(ctx_h, axis=1).astype(BF16)
            acc.append(lax.dot_general(ctx_b, wov, (((1,), (0,)), ((), ())),
                                       preferred_element_type=F32))

        ROWS = (B * SQ) // N_DEV
        for b in range(B):
            pstage_ref[b] = acc[b].astype(BF16)

        def _slice_for(t):
            return pstage_ref.at[t // (SQ // ROWS),
                                 pl.ds((t % (SQ // ROWS)) * ROWS, ROWS)]

        locrs = pltpu.make_async_copy(_slice_for(my), rsbuf_ref.at[my],
                                      loc_sem.at[0])
        locrs.start()
        for d in range(1, N_DEV):
            t = (my + d) % N_DEV
            pltpu.make_async_remote_copy(
                src_ref=_slice_for(t),
                dst_ref=rsbuf_ref.at[my],
                send_sem=send_rs.at[d],
                recv_sem=recv_rs.at[my],
                device_id=(t,),
                device_id_type=pl.DeviceIdType.MESH,
            ).start()
        locrs.wait()
        for d in range(1, N_DEV):
            s = (my - d + N_DEV) % N_DEV
            pltpu.make_async_remote_copy(
                src_ref=rsbuf_ref.at[s],
                dst_ref=rsbuf_ref.at[s],
                send_sem=send_rs.at[d],
                recv_sem=recv_rs.at[s],
                device_id=(s,),
                device_id_type=pl.DeviceIdType.MESH,
            ).wait_recv()

        agstage_ref[...] = jnp.sum(rsbuf_ref[...].astype(F32),
                                   axis=0).astype(BF16)
        locag = pltpu.make_async_copy(agstage_ref, agbuf_ref.at[my],
                                      loc_sem.at[1])
        locag.start()
        for d in range(1, N_DEV):
            t = (my + d) % N_DEV
            pltpu.make_async_remote_copy(
                src_ref=agstage_ref,
                dst_ref=agbuf_ref.at[my],
                send_sem=send_ag.at[d],
                recv_sem=recv_ag.at[my],
                device_id=(t,),
                device_id_type=pl.DeviceIdType.MESH,
            ).start()
        locag.wait()
        for d in range(1, N_DEV):
            s = (my - d + N_DEV) % N_DEV
            pltpu.make_async_remote_copy(
                src_ref=agstage_ref,
                dst_ref=agbuf_ref.at[s],
                send_sem=send_ag.at[d],
                recv_sem=recv_ag.at[s],
                device_id=(s,),
                device_id_type=pl.DeviceIdType.MESH,
            ).wait_recv()

        full = agbuf_ref[...].astype(F32).reshape(B * SQ, DM)
        for b in range(B):
            out_ref[b] = full[b * SQ:(b + 1) * SQ]

        for d in range(1, N_DEV):
            t = (my + d) % N_DEV
            pltpu.make_async_remote_copy(
                src_ref=_slice_for(t),
                dst_ref=rsbuf_ref.at[my],
                send_sem=send_rs.at[d],
                recv_sem=recv_rs.at[my],
                device_id=(t,),
                device_id_type=pl.DeviceIdType.MESH,
            ).wait_send()
            pltpu.make_async_remote_copy(
                src_ref=agstage_ref,
                dst_ref=agbuf_ref.at[my],
                send_sem=send_ag.at[d],
                recv_sem=recv_ag.at[my],
                device_id=(t,),
                device_id_type=pl.DeviceIdType.MESH,
            ).wait_send()

        for j in range(2):
            blk = 2 * my + j

            @pl.when(blk_needed(blk))
            def _(j=j, blk=blk):
                for d in range(1, N_DEV):
                    t = (my + d) % N_DEV
                    pltpu.make_async_remote_copy(
                        src_ref=kvbf_ref.at[:, :, pl.ds(j * BLK, BLK),
                                            pl.ds(t * GRP, GRP)],
                        dst_ref=kvbuf_ref.at[:, :, pl.ds(blk_dstoff(blk),
                                                         BLK), :],
                        send_sem=send1.at[2 * d + j],
                        recv_sem=recv1.at[2 * my + j],
                        device_id=(t,),
                        device_id_type=pl.DeviceIdType.MESH,
                    ).wait_send()

    return pl.pallas_call(
        body,
        out_shape=jax.ShapeDtypeStruct((B, SQ, DM), F32),
        in_specs=[pl.BlockSpec(memory_space=pltpu.VMEM)] * 5,
        out_specs=pl.BlockSpec(memory_space=pltpu.VMEM),
        scratch_shapes=[
            pltpu.VMEM((2, B, SKV_LOC, HQ * DH), BF16),
            pltpu.VMEM((2, B, SKVC, GRP), BF16),
            pltpu.VMEM((B, SQ, DM), BF16),
            pltpu.VMEM((N_DEV, (B * SQ) // N_DEV, DM), BF16),
            pltpu.VMEM(((B * SQ) // N_DEV, DM), BF16),
            pltpu.VMEM((N_DEV, (B * SQ) // N_DEV, DM), BF16),
            pltpu.SemaphoreType.DMA((2 * N_DEV,)),
            pltpu.SemaphoreType.DMA((2 * N_DEV,)),
            pltpu.SemaphoreType.DMA((N_DEV,)),
            pltpu.SemaphoreType.DMA((N_DEV,)),
            pltpu.SemaphoreType.DMA((N_DEV,)),
            pltpu.SemaphoreType.DMA((N_DEV,)),
            pltpu.SemaphoreType.DMA((2,)),
        ],
        compiler_params=pltpu.CompilerParams(collective_id=0),
    )(x, Wq, K_ext, V_ext, Wo)
